# baseline (speedup 1.0000x reference)
import functools

import jax
import jax.numpy as jnp
from jax import lax
from jax.experimental import pallas as pl
from jax.experimental.pallas import tpu as pltpu
from jax.experimental.pallas import tpu_sc as plsc

_NUM_CHARTS = 32
_ENC_DIM = 16
_ENC_H = 256
_ENC_W = 256
_GRID_H = 16
_GRID_W = 4096

_Q_PER_CHART = _GRID_H * _GRID_W
_TOTAL_Q = _NUM_CHARTS * _Q_PER_CHART
_LANES = 16
_CHUNK = 128
_SUPER = 2048
_N_SUPER = _Q_PER_CHART // _SUPER
_N_CHUNK = _SUPER // _CHUNK

def _sample_body(table, ux, uy, out, xv, yv, idxv, wv, rows, outv,
                 sem0, sem1):
    cid = lax.axis_index("c")
    sid = lax.axis_index("s")
    wid = sid * 2 + cid
    chart_base = wid * (_ENC_H * _ENC_W)
    qchart = wid * _Q_PER_CHART
    sems = (sem0, sem1)

    def compute_fire(s, b):
        cbase = s * _CHUNK
        for j in range(_CHUNK // _LANES):
            sl = pl.ds(cbase + j * _LANES, _LANES)
            x = xv[sl]
            y = yv[sl]
            fix = jnp.clip(((x + 1.0) * float(_ENC_W) - 1.0) * 0.5,
                           0.0, float(_ENC_W - 1))
            fiy = jnp.clip(((y + 1.0) * float(_ENC_H) - 1.0) * 0.5,
                           0.0, float(_ENC_H - 1))
            x0 = fix.astype(jnp.int32)
            y0 = fiy.astype(jnp.int32)
            fx = fix - x0.astype(jnp.float32)
            fy = fiy - y0.astype(jnp.float32)
            x1 = jnp.minimum(x0 + 1, _ENC_W - 1)
            y1 = jnp.minimum(y0 + 1, _ENC_H - 1)
            r0 = chart_base + y0 * _ENC_W
            r1 = chart_base + y1 * _ENC_W
            wsl = pl.ds(j * _LANES, _LANES)
            idxv[b, pl.ds(0 * _CHUNK + j * _LANES, _LANES)] = r0 + x0
            idxv[b, pl.ds(1 * _CHUNK + j * _LANES, _LANES)] = r1 + x0
            gx = 1.0 - fx
            gy = 1.0 - fy
            wv[b, 0, wsl] = gy * gx
            wv[b, 1, wsl] = gy * fx
            wv[b, 2, wsl] = fy * gx
            wv[b, 3, wsl] = fy * fx
        pltpu.async_copy(table.at[idxv.at[b]], rows.at[b], sems[b])

    def wait_blend(s, b):
        cbase = s * _CHUNK
        pltpu.make_async_copy(
            table.at[idxv.at[b]], rows.at[b], sems[b]).wait()
        for j in range(_CHUNK // _LANES):
            w0 = wv[b, 0, pl.ds(j * _LANES, _LANES)]
            w1 = wv[b, 1, pl.ds(j * _LANES, _LANES)]
            w2 = wv[b, 2, pl.ds(j * _LANES, _LANES)]
            w3 = wv[b, 3, pl.ds(j * _LANES, _LANES)]
            for t in range(_LANES):
                q = j * _LANES + t
                v00, v01 = plsc.unpack(rows[b, 0 * _CHUNK + q],
                                       format=plsc.PackFormat.INTERLEAVED)
                v10, v11 = plsc.unpack(rows[b, 1 * _CHUNK + q],
                                       format=plsc.PackFormat.INTERLEAVED)
                acc = (v00 * w0[t] + v01 * w1[t]
                       + v10 * w2[t] + v11 * w3[t])
                outv[cbase + q] = acc

    def super_body(g, carry):
        qbase = qchart + g * _SUPER
        pltpu.sync_copy(ux.at[pl.ds(qbase, _SUPER)], xv)
        pltpu.sync_copy(uy.at[pl.ds(qbase, _SUPER)], yv)
        compute_fire(0, 0)

        def pair_body(i, carry2):
            s0 = 2 * i
            compute_fire(s0 + 1, 1)
            wait_blend(s0, 0)
            compute_fire(s0 + 2, 0)
            wait_blend(s0 + 1, 1)
            return carry2

        lax.fori_loop(0, _N_CHUNK // 2 - 1, pair_body, 0)
        compute_fire(_N_CHUNK - 1, 1)
        wait_blend(_N_CHUNK - 2, 0)
        wait_blend(_N_CHUNK - 1, 1)
        pltpu.sync_copy(outv, out.at[pl.ds(qbase, _SUPER)])
        return carry

    lax.fori_loop(0, _N_SUPER, super_body, 0)


_sample = functools.partial(
    pl.kernel,
    out_type=jax.ShapeDtypeStruct((_TOTAL_Q, _ENC_DIM), jnp.float32),
    mesh=plsc.VectorSubcoreMesh(core_axis_name="c", subcore_axis_name="s"),
    compiler_params=pltpu.CompilerParams(use_tc_tiling_on_sc=False,
                                         needs_layout_passes=False),
    scratch_types=[
        pltpu.VMEM((_SUPER,), jnp.float32),
        pltpu.VMEM((_SUPER,), jnp.float32),
        pltpu.VMEM((2, 2 * _CHUNK), jnp.int32),
        pltpu.VMEM((2, 4, _CHUNK), jnp.float32),
        pltpu.VMEM((2, 2 * _CHUNK, 2 * _ENC_DIM), jnp.bfloat16),
        pltpu.VMEM((_SUPER, _ENC_DIM), jnp.float32),
        pltpu.SemaphoreType.DMA,
        pltpu.SemaphoreType.DMA,
    ],
)(_sample_body)


def kernel(uv, encodings):
    tab = jnp.transpose(encodings, (0, 2, 3, 1)).reshape(
        _NUM_CHARTS * _ENC_H * _ENC_W, _ENC_DIM)
    tab_next = jnp.concatenate([tab[1:], tab[-1:]], axis=0)
    table = jnp.stack([tab, tab_next], axis=2).reshape(
        _NUM_CHARTS * _ENC_H * _ENC_W, 2 * _ENC_DIM).astype(jnp.bfloat16)
    uvf = uv.reshape(_TOTAL_Q, 2)
    out = _sample(table, uvf[:, 0], uvf[:, 1])
    return out.reshape(_NUM_CHARTS, _GRID_H, _GRID_W, _ENC_DIM)


# speedup vs baseline: 2.0621x; 2.0621x over previous
import functools

import jax
import jax.numpy as jnp
from jax import lax
from jax.experimental import pallas as pl
from jax.experimental.pallas import tpu as pltpu
from jax.experimental.pallas import tpu_sc as plsc

_NUM_CHARTS = 32
_ENC_DIM = 16
_ENC_H = 256
_ENC_W = 256
_GRID_H = 16
_GRID_W = 4096

_Q_PER_CHART = _GRID_H * _GRID_W
_TOTAL_Q = _NUM_CHARTS * _Q_PER_CHART
_LANES = 16
_CHUNK = 128
_SUPER = 2048
_N_SUPER = _Q_PER_CHART // _SUPER
_N_CHUNK = _SUPER // _CHUNK

def _sample_body(table, ux, uy, out, xv, yv, idxv, wv, rows, outv,
                 sem0, sem1):
    cid = lax.axis_index("c")
    sid = lax.axis_index("s")
    wid = sid * 2 + cid
    chart_base = wid * (_ENC_H * _ENC_W)
    qchart = wid * _Q_PER_CHART
    sems = (sem0, sem1)

    def compute_fire(s, b):
        cbase = s * _CHUNK
        for j in range(_CHUNK // _LANES):
            sl = pl.ds(cbase + j * _LANES, _LANES)
            x = xv[sl]
            y = yv[sl]
            fix = jnp.clip(((x + 1.0) * float(_ENC_W) - 1.0) * 0.5,
                           0.0, float(_ENC_W - 1))
            fiy = jnp.clip(((y + 1.0) * float(_ENC_H) - 1.0) * 0.5,
                           0.0, float(_ENC_H - 1))
            x0 = fix.astype(jnp.int32)
            y0 = fiy.astype(jnp.int32)
            fx = fix - x0.astype(jnp.float32)
            fy = fiy - y0.astype(jnp.float32)
            x1 = jnp.minimum(x0 + 1, _ENC_W - 1)
            y1 = jnp.minimum(y0 + 1, _ENC_H - 1)
            r0 = chart_base + y0 * _ENC_W
            r1 = chart_base + y1 * _ENC_W
            wsl = pl.ds(j * _LANES, _LANES)
            idxv[b, pl.ds(0 * _CHUNK + j * _LANES, _LANES)] = r0 + x0
            idxv[b, pl.ds(1 * _CHUNK + j * _LANES, _LANES)] = r0 + x1
            idxv[b, pl.ds(2 * _CHUNK + j * _LANES, _LANES)] = r1 + x0
            idxv[b, pl.ds(3 * _CHUNK + j * _LANES, _LANES)] = r1 + x1
            gx = 1.0 - fx
            gy = 1.0 - fy
            wv[b, 0, wsl] = gy * gx
            wv[b, 1, wsl] = gy * fx
            wv[b, 2, wsl] = fy * gx
            wv[b, 3, wsl] = fy * fx
        pltpu.async_copy(table.at[idxv.at[b]], rows.at[b], sems[b])

    def wait_blend(s, b):
        cbase = s * _CHUNK
        pltpu.make_async_copy(
            table.at[idxv.at[b]], rows.at[b], sems[b]).wait()
        for j in range(_CHUNK // _LANES):
            w0 = wv[b, 0, pl.ds(j * _LANES, _LANES)]
            w1 = wv[b, 1, pl.ds(j * _LANES, _LANES)]
            w2 = wv[b, 2, pl.ds(j * _LANES, _LANES)]
            w3 = wv[b, 3, pl.ds(j * _LANES, _LANES)]
            for t in range(_LANES):
                q = j * _LANES + t
                acc = ((rows[b, 0 * _CHUNK + q] * w0[t]
                        + rows[b, 1 * _CHUNK + q] * w1[t])
                       + (rows[b, 2 * _CHUNK + q] * w2[t]
                          + rows[b, 3 * _CHUNK + q] * w3[t]))
                outv[cbase + q] = acc

    def super_body(g, carry):
        qbase = qchart + g * _SUPER
        pltpu.sync_copy(ux.at[pl.ds(qbase, _SUPER)], xv)
        pltpu.sync_copy(uy.at[pl.ds(qbase, _SUPER)], yv)
        compute_fire(0, 0)

        def pair_body(i, carry2):
            s0 = 2 * i
            compute_fire(s0 + 1, 1)
            wait_blend(s0, 0)
            compute_fire(s0 + 2, 0)
            wait_blend(s0 + 1, 1)
            return carry2

        lax.fori_loop(0, _N_CHUNK // 2 - 1, pair_body, 0)
        compute_fire(_N_CHUNK - 1, 1)
        wait_blend(_N_CHUNK - 2, 0)
        wait_blend(_N_CHUNK - 1, 1)
        pltpu.sync_copy(outv, out.at[pl.ds(qbase, _SUPER)])
        return carry

    lax.fori_loop(0, _N_SUPER, super_body, 0)


_sample = functools.partial(
    pl.kernel,
    out_type=jax.ShapeDtypeStruct((_TOTAL_Q, _ENC_DIM), jnp.float32),
    mesh=plsc.VectorSubcoreMesh(core_axis_name="c", subcore_axis_name="s"),
    compiler_params=pltpu.CompilerParams(use_tc_tiling_on_sc=False,
                                         needs_layout_passes=False),
    scratch_types=[
        pltpu.VMEM((_SUPER,), jnp.float32),
        pltpu.VMEM((_SUPER,), jnp.float32),
        pltpu.VMEM((2, 4 * _CHUNK), jnp.int32),
        pltpu.VMEM((2, 4, _CHUNK), jnp.float32),
        pltpu.VMEM((2, 4 * _CHUNK, _ENC_DIM), jnp.float32),
        pltpu.VMEM((_SUPER, _ENC_DIM), jnp.float32),
        pltpu.SemaphoreType.DMA,
        pltpu.SemaphoreType.DMA,
    ],
)(_sample_body)


def kernel(uv, encodings):
    table = jnp.transpose(encodings, (0, 2, 3, 1)).reshape(
        _NUM_CHARTS * _ENC_H * _ENC_W, _ENC_DIM)
    uvf = uv.reshape(_TOTAL_Q, 2)
    out = _sample(table, uvf[:, 0], uvf[:, 1])
    return out.reshape(_NUM_CHARTS, _GRID_H, _GRID_W, _ENC_DIM)


# speedup vs baseline: 2.2350x; 1.0839x over previous
import functools

import jax
import jax.numpy as jnp
from jax import lax
from jax.experimental import pallas as pl
from jax.experimental.pallas import tpu as pltpu
from jax.experimental.pallas import tpu_sc as plsc

_NUM_CHARTS = 32
_ENC_DIM = 16
_ENC_H = 256
_ENC_W = 256
_GRID_H = 16
_GRID_W = 4096

_Q_PER_CHART = _GRID_H * _GRID_W
_TOTAL_Q = _NUM_CHARTS * _Q_PER_CHART
_LANES = 16
_CHUNK = 128
_SUPER = 4096
_N_SUPER = _Q_PER_CHART // _SUPER
_N_CHUNK = _SUPER // _CHUNK

def _sample_body(table, ux, uy, out, xv, yv, idxv, wv, rows, outv,
                 sem0, sem1):
    cid = lax.axis_index("c")
    sid = lax.axis_index("s")
    wid = sid * 2 + cid
    chart_base = wid * (_ENC_H * _ENC_W)
    qchart = wid * _Q_PER_CHART
    sems = (sem0, sem1)

    def compute_fire(s, b):
        cbase = s * _CHUNK
        for j in range(_CHUNK // _LANES):
            sl = pl.ds(cbase + j * _LANES, _LANES)
            x = xv[sl]
            y = yv[sl]
            fix = jnp.clip(((x + 1.0) * float(_ENC_W) - 1.0) * 0.5,
                           0.0, float(_ENC_W - 1))
            fiy = jnp.clip(((y + 1.0) * float(_ENC_H) - 1.0) * 0.5,
                           0.0, float(_ENC_H - 1))
            x0 = fix.astype(jnp.int32)
            y0 = fiy.astype(jnp.int32)
            fx = fix - x0.astype(jnp.float32)
            fy = fiy - y0.astype(jnp.float32)
            x1 = jnp.minimum(x0 + 1, _ENC_W - 1)
            y1 = jnp.minimum(y0 + 1, _ENC_H - 1)
            r0 = chart_base + y0 * _ENC_W
            r1 = chart_base + y1 * _ENC_W
            wsl = pl.ds(j * _LANES, _LANES)
            idxv[b, pl.ds(0 * _CHUNK + j * _LANES, _LANES)] = r0 + x0
            idxv[b, pl.ds(1 * _CHUNK + j * _LANES, _LANES)] = r0 + x1
            idxv[b, pl.ds(2 * _CHUNK + j * _LANES, _LANES)] = r1 + x0
            idxv[b, pl.ds(3 * _CHUNK + j * _LANES, _LANES)] = r1 + x1
            gx = 1.0 - fx
            gy = 1.0 - fy
            wv[b, 0, wsl] = gy * gx
            wv[b, 1, wsl] = gy * fx
            wv[b, 2, wsl] = fy * gx
            wv[b, 3, wsl] = fy * fx
        pltpu.async_copy(table.at[idxv.at[b]], rows.at[b], sems[b])

    def wait_blend(s, b):
        cbase = s * _CHUNK
        pltpu.make_async_copy(
            table.at[idxv.at[b]], rows.at[b], sems[b]).wait()
        for j in range(_CHUNK // _LANES):
            w0 = wv[b, 0, pl.ds(j * _LANES, _LANES)]
            w1 = wv[b, 1, pl.ds(j * _LANES, _LANES)]
            w2 = wv[b, 2, pl.ds(j * _LANES, _LANES)]
            w3 = wv[b, 3, pl.ds(j * _LANES, _LANES)]
            for t in range(_LANES):
                q = j * _LANES + t
                acc = (rows[b, 0 * _CHUNK + q] * w0[t]
                       + rows[b, 1 * _CHUNK + q] * w1[t]
                       + rows[b, 2 * _CHUNK + q] * w2[t]
                       + rows[b, 3 * _CHUNK + q] * w3[t])
                outv[cbase + q] = acc

    def super_body(g, carry):
        qbase = qchart + g * _SUPER
        pltpu.sync_copy(ux.at[pl.ds(qbase, _SUPER)], xv)
        pltpu.sync_copy(uy.at[pl.ds(qbase, _SUPER)], yv)
        compute_fire(0, 0)

        def pair_body(i, carry2):
            s0 = 2 * i
            compute_fire(s0 + 1, 1)
            wait_blend(s0, 0)
            compute_fire(s0 + 2, 0)
            wait_blend(s0 + 1, 1)
            return carry2

        lax.fori_loop(0, _N_CHUNK // 2 - 1, pair_body, 0)
        compute_fire(_N_CHUNK - 1, 1)
        wait_blend(_N_CHUNK - 2, 0)
        wait_blend(_N_CHUNK - 1, 1)
        pltpu.sync_copy(outv, out.at[pl.ds(qbase, _SUPER)])
        return carry

    lax.fori_loop(0, _N_SUPER, super_body, 0)


_sample = functools.partial(
    pl.kernel,
    out_type=jax.ShapeDtypeStruct((_TOTAL_Q, _ENC_DIM), jnp.float32),
    mesh=plsc.VectorSubcoreMesh(core_axis_name="c", subcore_axis_name="s"),
    compiler_params=pltpu.CompilerParams(use_tc_tiling_on_sc=False,
                                         needs_layout_passes=False),
    scratch_types=[
        pltpu.VMEM((_SUPER,), jnp.float32),
        pltpu.VMEM((_SUPER,), jnp.float32),
        pltpu.VMEM((2, 4 * _CHUNK), jnp.int32),
        pltpu.VMEM((2, 4, _CHUNK), jnp.float32),
        pltpu.VMEM((2, 4 * _CHUNK, _ENC_DIM), jnp.float32),
        pltpu.VMEM((_SUPER, _ENC_DIM), jnp.float32),
        pltpu.SemaphoreType.DMA,
        pltpu.SemaphoreType.DMA,
    ],
)(_sample_body)


def kernel(uv, encodings):
    table = jnp.transpose(encodings, (0, 2, 3, 1)).reshape(
        _NUM_CHARTS * _ENC_H * _ENC_W, _ENC_DIM)
    uvf = uv.reshape(_TOTAL_Q, 2)
    out = _sample(table, uvf[:, 0], uvf[:, 1])
    return out.reshape(_NUM_CHARTS, _GRID_H, _GRID_W, _ENC_DIM)
